# Initial kernel scaffold; baseline (speedup 1.0000x reference)
import jax
import jax.numpy as jnp
from jax.experimental import pallas as pl


def kernel(node_orc, edge_index, W1, b1, W2, b2, gamma, beta):
    raise NotImplementedError("write your pallas kernel here")


# speedup vs baseline: 27.5938x; 27.5938x over previous
import functools
import math

import jax
import jax.numpy as jnp
from jax import lax
from jax.experimental import pallas as pl
from jax.experimental.pallas import tpu as pltpu
from jax.experimental.pallas import tpu_sc as plsc

N = 10000
E = 160000
DC = 16
HID = 32

NC = 2
NS = 16
NW = NC * NS
LANES = 16

NP = 10240
ET = 5008
EP = ET * NW


def _sc_scatter(orc_p, u_p, v_p):
    mesh = plsc.VectorSubcoreMesh(core_axis_name="c", subcore_axis_name="s")

    @functools.partial(
        pl.kernel,
        out_type=(
            jax.ShapeDtypeStruct((NW, NP), jnp.float32),
            jax.ShapeDtypeStruct((NW, NP), jnp.float32),
        ),
        mesh=mesh,
        compiler_params=pltpu.CompilerParams(
            use_tc_tiling_on_sc=False, needs_layout_passes=False),
        scratch_types=(
            pltpu.VMEM((NP,), jnp.float32),
            pltpu.VMEM((ET,), jnp.int32),
            pltpu.VMEM((ET,), jnp.int32),
            pltpu.VMEM((NP,), jnp.float32),
            pltpu.VMEM((NP,), jnp.float32),
        ),
    )
    def sc_kernel(orc_hbm, u_hbm, v_hbm, msum_hbm, cnt_hbm,
                  orc_v, u_v, v_v, ms_v, cn_v):
        wid = lax.axis_index("s") * NC + lax.axis_index("c")
        base = wid * ET
        pltpu.sync_copy(orc_hbm, orc_v)
        pltpu.sync_copy(u_hbm.at[pl.ds(base, ET)], u_v)
        pltpu.sync_copy(v_hbm.at[pl.ds(base, ET)], v_v)

        zeros = jnp.zeros((LANES,), jnp.float32)

        def zero_body(i, carry):
            ms_v[pl.ds(i * LANES, LANES)] = zeros
            cn_v[pl.ds(i * LANES, LANES)] = zeros
            return carry

        lax.fori_loop(0, NP // LANES, zero_body, 0)

        ones = jnp.ones((LANES,), jnp.float32)

        def body(i, carry):
            off = i * LANES
            u16 = u_v[pl.ds(off, LANES)]
            v16 = v_v[pl.ds(off, LANES)]
            ou = plsc.load_gather(orc_v, [u16])
            ov = plsc.load_gather(orc_v, [v16])
            plsc.addupdate_scatter(ms_v, [u16], ov)
            plsc.addupdate_scatter(ms_v, [v16], ou)
            plsc.addupdate_scatter(cn_v, [u16], ones)
            plsc.addupdate_scatter(cn_v, [v16], ones)
            return carry

        lax.fori_loop(0, ET // LANES, body, 0)

        pltpu.sync_copy(ms_v, msum_hbm.at[wid])
        pltpu.sync_copy(cn_v, cnt_hbm.at[wid])

    return sc_kernel(orc_p, u_p, v_p)


def _tc_body(orc_ref, ms_ref, cn_ref, w1_ref, b1_ref, w2_ref, b2_ref,
             g_ref, be_ref, out_ref):
    f32 = jnp.float32
    ms = ms_ref[...]
    cn = cn_ref[...]
    mc = jnp.concatenate([ms, cn], axis=0)
    row = lax.broadcasted_iota(jnp.int32, (2 * NW, 2), 0)
    col = lax.broadcasted_iota(jnp.int32, (2 * NW, 2), 1)
    sel = ((row // NW) == col).astype(f32)
    both = lax.dot_general(mc, sel, (((0,), (0,)), ((), ())),
                           preferred_element_type=f32)
    msum = both[:, 0:1]
    cnt = both[:, 1:2]

    nm = jnp.where(cnt > 0, msum / jnp.where(cnt > 0, cnt, 1.0), 0.0)
    orc = orc_ref[...]
    scale = 1.0 / (2.0 + 1e-8)
    no = jnp.clip((orc + 1.0) * scale, 0.0, 1.0)
    nn = jnp.clip((nm + 1.0) * scale, 0.0, 1.0)

    j = lax.broadcasted_iota(jnp.int32, (1, DC), 1)
    base = jnp.where(j < DC // 2, no, nn)
    freq = (((j % (DC // 2)) // 2) + 1).astype(f32) * math.pi
    ang = base * freq
    phi = jnp.where(j % 2 == 0, jnp.sin(ang), jnp.cos(ang))

    h = lax.dot_general(phi, w1_ref[...], (((1,), (1,)), ((), ())),
                        preferred_element_type=f32) + b1_ref[...]
    h = jnp.maximum(h, 0.0)
    h2 = lax.dot_general(h, w2_ref[...], (((1,), (1,)), ((), ())),
                         preferred_element_type=f32) + b2_ref[...]
    mu = jnp.mean(h2, axis=1, keepdims=True)
    d = h2 - mu
    var = jnp.mean(d * d, axis=1, keepdims=True)
    ln = d / jnp.sqrt(var + 1e-5) * g_ref[...] + be_ref[...]
    out_ref[...] = ln + phi


def _tc_dense(orc2d, msum_p, cnt_p, W1, b1, W2, b2, gamma, beta,
              interpret=False):
    C = 1024
    return pl.pallas_call(
        _tc_body,
        grid=(NP // C,),
        in_specs=[
            pl.BlockSpec((C, 1), lambda i: (i, 0)),
            pl.BlockSpec((NW, C), lambda i: (0, i)),
            pl.BlockSpec((NW, C), lambda i: (0, i)),
            pl.BlockSpec((HID, DC), lambda i: (0, 0)),
            pl.BlockSpec((1, HID), lambda i: (0, 0)),
            pl.BlockSpec((DC, HID), lambda i: (0, 0)),
            pl.BlockSpec((1, DC), lambda i: (0, 0)),
            pl.BlockSpec((1, DC), lambda i: (0, 0)),
            pl.BlockSpec((1, DC), lambda i: (0, 0)),
        ],
        out_specs=pl.BlockSpec((C, DC), lambda i: (i, 0)),
        out_shape=jax.ShapeDtypeStruct((NP, DC), jnp.float32),
        interpret=interpret,
    )(orc2d, msum_p, cnt_p, W1, b1, W2, b2, gamma, beta)


def kernel(node_orc, edge_index, W1, b1, W2, b2, gamma, beta):
    orc_p = jnp.pad(node_orc, (0, NP - N))
    u_p = jnp.pad(edge_index[0], (0, EP - E), constant_values=N)
    v_p = jnp.pad(edge_index[1], (0, EP - E), constant_values=N)
    msum_p, cnt_p = _sc_scatter(orc_p, u_p, v_p)
    out = _tc_dense(orc_p.reshape(NP, 1), msum_p, cnt_p,
                    W1, b1.reshape(1, HID), W2, b2.reshape(1, DC),
                    gamma.reshape(1, DC), beta.reshape(1, DC))
    return out[:N]


# speedup vs baseline: 30.6762x; 1.1117x over previous
import functools
import math

import jax
import jax.numpy as jnp
from jax import lax
from jax.experimental import pallas as pl
from jax.experimental.pallas import tpu as pltpu
from jax.experimental.pallas import tpu_sc as plsc

N = 10000
E = 160000
DC = 16
HID = 32

NC = 2
NS = 16
NW = NC * NS
LANES = 16

NP = 10240
CHUNK = E // NW
FULL = CHUNK // LANES
TAIL = CHUNK - FULL * LANES
CCAP = (FULL + 1) * LANES


def _sc_scatter(orc, ei_flat):
    mesh = plsc.VectorSubcoreMesh(core_axis_name="c", subcore_axis_name="s")

    @functools.partial(
        pl.kernel,
        out_type=(
            jax.ShapeDtypeStruct((NW, NP), jnp.float32),
            jax.ShapeDtypeStruct((NW, NP), jnp.float32),
        ),
        mesh=mesh,
        compiler_params=pltpu.CompilerParams(
            use_tc_tiling_on_sc=False, needs_layout_passes=False),
        scratch_types=(
            pltpu.VMEM((NP,), jnp.float32),
            pltpu.VMEM((CCAP,), jnp.int32),
            pltpu.VMEM((CCAP,), jnp.int32),
            pltpu.VMEM((NP,), jnp.float32),
            pltpu.VMEM((NP,), jnp.float32),
        ),
    )
    def sc_kernel(orc_hbm, ei_hbm, msum_hbm, cnt_hbm,
                  orc_v, u_v, v_v, ms_v, cn_v):
        wid = lax.axis_index("s") * NC + lax.axis_index("c")
        base = wid * CHUNK
        junk = jnp.full((LANES,), N, jnp.int32)
        u_v[pl.ds(FULL * LANES, LANES)] = junk
        v_v[pl.ds(FULL * LANES, LANES)] = junk
        pltpu.sync_copy(orc_hbm, orc_v.at[pl.ds(0, N)])
        pltpu.sync_copy(ei_hbm.at[pl.ds(base, CHUNK)], u_v.at[pl.ds(0, CHUNK)])
        pltpu.sync_copy(ei_hbm.at[pl.ds(E + base, CHUNK)],
                        v_v.at[pl.ds(0, CHUNK)])

        zeros = jnp.zeros((LANES,), jnp.float32)

        def zero_body(i, carry):
            ms_v[pl.ds(i * LANES, LANES)] = zeros
            cn_v[pl.ds(i * LANES, LANES)] = zeros
            return carry

        lax.fori_loop(0, NP // LANES, zero_body, 0)

        ones = jnp.ones((LANES,), jnp.float32)

        def body(i, carry):
            off = i * LANES
            u16 = u_v[pl.ds(off, LANES)]
            v16 = v_v[pl.ds(off, LANES)]
            ou = plsc.load_gather(orc_v, [u16])
            ov = plsc.load_gather(orc_v, [v16])
            plsc.addupdate_scatter(ms_v, [u16], ov)
            plsc.addupdate_scatter(ms_v, [v16], ou)
            plsc.addupdate_scatter(cn_v, [u16], ones)
            plsc.addupdate_scatter(cn_v, [v16], ones)
            return carry

        lax.fori_loop(0, FULL + 1, body, 0)

        pltpu.sync_copy(ms_v, msum_hbm.at[wid])
        pltpu.sync_copy(cn_v, cnt_hbm.at[wid])

    return sc_kernel(orc, ei_flat)


def _tc_body(orc_ref, ms_ref, cn_ref, w1_ref, b1_ref, w2_ref, b2_ref,
             g_ref, be_ref, out_ref):
    f32 = jnp.float32
    ms = ms_ref[...]
    cn = cn_ref[...]
    mc = jnp.concatenate([ms, cn], axis=0)
    row = lax.broadcasted_iota(jnp.int32, (2 * NW, 2), 0)
    col = lax.broadcasted_iota(jnp.int32, (2 * NW, 2), 1)
    sel = ((row // NW) == col).astype(f32)
    both = lax.dot_general(mc, sel, (((0,), (0,)), ((), ())),
                           preferred_element_type=f32)
    msum = both[:, 0:1]
    cnt = both[:, 1:2]

    nm = jnp.where(cnt > 0, msum / jnp.where(cnt > 0, cnt, 1.0), 0.0)
    orc = orc_ref[...]
    scale = 1.0 / (2.0 + 1e-8)
    no = jnp.clip((orc + 1.0) * scale, 0.0, 1.0)
    nn = jnp.clip((nm + 1.0) * scale, 0.0, 1.0)

    j = lax.broadcasted_iota(jnp.int32, (1, DC), 1)
    base = jnp.where(j < DC // 2, no, nn)
    freq = (((j % (DC // 2)) // 2) + 1).astype(f32) * math.pi
    ang = base * freq
    phi = jnp.where(j % 2 == 0, jnp.sin(ang), jnp.cos(ang))

    h = lax.dot_general(phi, w1_ref[...], (((1,), (1,)), ((), ())),
                        preferred_element_type=f32) + b1_ref[...]
    h = jnp.maximum(h, 0.0)
    h2 = lax.dot_general(h, w2_ref[...], (((1,), (1,)), ((), ())),
                         preferred_element_type=f32) + b2_ref[...]
    mu = jnp.mean(h2, axis=1, keepdims=True)
    d = h2 - mu
    var = jnp.mean(d * d, axis=1, keepdims=True)
    ln = d / jnp.sqrt(var + 1e-5) * g_ref[...] + be_ref[...]
    out_ref[...] = ln + phi


def _tc_dense(orc2d, msum_p, cnt_p, W1, b1, W2, b2, gamma, beta,
              interpret=False):
    C = 1024
    return pl.pallas_call(
        _tc_body,
        grid=(NP // C,),
        in_specs=[
            pl.BlockSpec((C, 1), lambda i: (i, 0)),
            pl.BlockSpec((NW, C), lambda i: (0, i)),
            pl.BlockSpec((NW, C), lambda i: (0, i)),
            pl.BlockSpec((HID, DC), lambda i: (0, 0)),
            pl.BlockSpec((1, HID), lambda i: (0, 0)),
            pl.BlockSpec((DC, HID), lambda i: (0, 0)),
            pl.BlockSpec((1, DC), lambda i: (0, 0)),
            pl.BlockSpec((1, DC), lambda i: (0, 0)),
            pl.BlockSpec((1, DC), lambda i: (0, 0)),
        ],
        out_specs=pl.BlockSpec((C, DC), lambda i: (i, 0)),
        out_shape=jax.ShapeDtypeStruct((N, DC), jnp.float32),
        interpret=interpret,
    )(orc2d, msum_p, cnt_p, W1, b1, W2, b2, gamma, beta)


def kernel(node_orc, edge_index, W1, b1, W2, b2, gamma, beta):
    msum_p, cnt_p = _sc_scatter(node_orc, edge_index.reshape(2 * E))
    return _tc_dense(node_orc.reshape(N, 1), msum_p, cnt_p,
                     W1, b1.reshape(1, HID), W2, b2.reshape(1, DC),
                     gamma.reshape(1, DC), beta.reshape(1, DC))


# speedup vs baseline: 45.7047x; 1.4899x over previous
import functools
import math

import jax
import jax.numpy as jnp
from jax import lax
from jax.experimental import pallas as pl
from jax.experimental.pallas import tpu as pltpu
from jax.experimental.pallas import tpu_sc as plsc

N = 10000
E = 160000
DC = 16
HID = 32

NC = 2
NS = 16
NW = NC * NS
LANES = 16

NP = 10240
CHUNK = E // NW
FULL = CHUNK // LANES
TAIL = CHUNK - FULL * LANES
CCAP = (FULL + 1) * LANES


def _sc_scatter(orc, ei_flat):
    mesh = plsc.VectorSubcoreMesh(core_axis_name="c", subcore_axis_name="s")

    @functools.partial(
        pl.kernel,
        out_type=(
            jax.ShapeDtypeStruct((NW, NP), jnp.float32),
            jax.ShapeDtypeStruct((NW, NP), jnp.float32),
        ),
        mesh=mesh,
        compiler_params=pltpu.CompilerParams(
            use_tc_tiling_on_sc=False, needs_layout_passes=False),
        scratch_types=(
            pltpu.VMEM((NP,), jnp.float32),
            pltpu.VMEM((CCAP,), jnp.int32),
            pltpu.VMEM((CCAP,), jnp.int32),
            pltpu.VMEM((NP,), jnp.float32),
            pltpu.VMEM((NP,), jnp.float32),
        ),
    )
    def sc_kernel(orc_hbm, ei_hbm, msum_hbm, cnt_hbm,
                  orc_v, u_v, v_v, ms_v, cn_v):
        wid = lax.axis_index("s") * NC + lax.axis_index("c")
        base = wid * CHUNK
        junk = jnp.full((LANES,), N, jnp.int32)
        u_v[pl.ds(FULL * LANES, LANES)] = junk
        v_v[pl.ds(FULL * LANES, LANES)] = junk
        pltpu.sync_copy(orc_hbm, orc_v.at[pl.ds(0, N)])
        pltpu.sync_copy(ei_hbm.at[pl.ds(base, CHUNK)], u_v.at[pl.ds(0, CHUNK)])
        pltpu.sync_copy(ei_hbm.at[pl.ds(E + base, CHUNK)],
                        v_v.at[pl.ds(0, CHUNK)])

        zeros = jnp.zeros((LANES,), jnp.float32)

        def zero_body(i, carry):
            ms_v[pl.ds(i * LANES, LANES)] = zeros
            cn_v[pl.ds(i * LANES, LANES)] = zeros
            return carry

        lax.fori_loop(0, NP // LANES, zero_body, 0)

        ones = jnp.ones((LANES,), jnp.float32)

        def body(i, carry):
            off = i * LANES
            u16 = u_v[pl.ds(off, LANES)]
            v16 = v_v[pl.ds(off, LANES)]
            ou = plsc.load_gather(orc_v, [u16])
            ov = plsc.load_gather(orc_v, [v16])
            plsc.addupdate_scatter(ms_v, [u16], ov)
            plsc.addupdate_scatter(ms_v, [v16], ou)
            plsc.addupdate_scatter(cn_v, [u16], ones)
            plsc.addupdate_scatter(cn_v, [v16], ones)
            return carry

        lax.fori_loop(0, FULL + 1, body, 0)

        pltpu.sync_copy(ms_v, msum_hbm.at[wid])
        pltpu.sync_copy(cn_v, cnt_hbm.at[wid])

    return sc_kernel(orc, ei_flat)


def _tc_body(orc_ref, ms_ref, cn_ref, w1_ref, b1_ref, w2_ref, b2_ref,
             g_ref, be_ref, out_ref):
    f32 = jnp.float32
    msum = jnp.sum(ms_ref[...], axis=0, keepdims=True)
    cnt = jnp.sum(cn_ref[...], axis=0, keepdims=True)
    nm = jnp.where(cnt > 0, msum / jnp.where(cnt > 0, cnt, 1.0), 0.0)
    orc = orc_ref[...]
    scale = 1.0 / (2.0 + 1e-8)
    no = jnp.clip((orc + 1.0) * scale, 0.0, 1.0)
    nn = jnp.clip((nm + 1.0) * scale, 0.0, 1.0)

    j = lax.broadcasted_iota(jnp.int32, (DC, 1), 0)
    base = jnp.where(j < DC // 2, no, nn)
    freq = (((j % (DC // 2)) // 2) + 1).astype(f32) * math.pi
    ang = base * freq
    phi = jnp.where(j % 2 == 0, jnp.sin(ang), jnp.cos(ang))

    h = lax.dot_general(w1_ref[...], phi, (((1,), (0,)), ((), ())),
                        preferred_element_type=f32) + b1_ref[...]
    h = jnp.maximum(h, 0.0)
    h2 = lax.dot_general(w2_ref[...], h, (((1,), (0,)), ((), ())),
                         preferred_element_type=f32) + b2_ref[...]
    mu = jnp.mean(h2, axis=0, keepdims=True)
    d = h2 - mu
    var = jnp.mean(d * d, axis=0, keepdims=True)
    ln = d / jnp.sqrt(var + 1e-5) * g_ref[...] + be_ref[...]
    outT = ln + phi
    eye = (lax.broadcasted_iota(jnp.int32, (DC, DC), 0)
           == lax.broadcasted_iota(jnp.int32, (DC, DC), 1)).astype(f32)
    out_ref[...] = lax.dot_general(outT, eye, (((0,), (0,)), ((), ())),
                                   preferred_element_type=f32)


def _tc_dense(orc_row, msum_p, cnt_p, W1, b1, W2, b2, gamma, beta,
              interpret=False):
    C = 1024
    return pl.pallas_call(
        _tc_body,
        grid=(NP // C,),
        in_specs=[
            pl.BlockSpec((1, C), lambda i: (0, i)),
            pl.BlockSpec((NW, C), lambda i: (0, i)),
            pl.BlockSpec((NW, C), lambda i: (0, i)),
            pl.BlockSpec((HID, DC), lambda i: (0, 0)),
            pl.BlockSpec((HID, 1), lambda i: (0, 0)),
            pl.BlockSpec((DC, HID), lambda i: (0, 0)),
            pl.BlockSpec((DC, 1), lambda i: (0, 0)),
            pl.BlockSpec((DC, 1), lambda i: (0, 0)),
            pl.BlockSpec((DC, 1), lambda i: (0, 0)),
        ],
        out_specs=pl.BlockSpec((C, DC), lambda i: (i, 0)),
        out_shape=jax.ShapeDtypeStruct((N, DC), jnp.float32),
        interpret=interpret,
    )(orc_row, msum_p, cnt_p, W1, b1, W2, b2, gamma, beta)


def kernel(node_orc, edge_index, W1, b1, W2, b2, gamma, beta):
    msum_p, cnt_p = _sc_scatter(node_orc, edge_index.reshape(2 * E))
    return _tc_dense(node_orc.reshape(1, N), msum_p, cnt_p,
                     W1, b1.reshape(HID, 1), W2, b2.reshape(DC, 1),
                     gamma.reshape(DC, 1), beta.reshape(DC, 1))


# speedup vs baseline: 48.2887x; 1.0565x over previous
import functools
import math

import jax
import jax.numpy as jnp
from jax import lax
from jax.experimental import pallas as pl
from jax.experimental.pallas import tpu as pltpu
from jax.experimental.pallas import tpu_sc as plsc

N = 10000
E = 160000
DC = 16
HID = 32

NC = 2
NS = 16
NW = NC * NS
LANES = 16

NP = 10240
CHUNK = E // NW
FULL = CHUNK // LANES
TAIL = CHUNK - FULL * LANES
CCAP = (FULL + 1) * LANES


def _sc_scatter(orc, ei_flat):
    mesh = plsc.VectorSubcoreMesh(core_axis_name="c", subcore_axis_name="s")

    @functools.partial(
        pl.kernel,
        out_type=(
            jax.ShapeDtypeStruct((NW, NP), jnp.float32),
            jax.ShapeDtypeStruct((NW, NP), jnp.float32),
        ),
        mesh=mesh,
        compiler_params=pltpu.CompilerParams(
            use_tc_tiling_on_sc=False, needs_layout_passes=False),
        scratch_types=(
            pltpu.VMEM((NP,), jnp.float32),
            pltpu.VMEM((CCAP,), jnp.int32),
            pltpu.VMEM((CCAP,), jnp.int32),
            pltpu.VMEM((NP,), jnp.float32),
            pltpu.VMEM((NP,), jnp.float32),
        ),
    )
    def sc_kernel(orc_hbm, ei_hbm, msum_hbm, cnt_hbm,
                  orc_v, u_v, v_v, ms_v, cn_v):
        wid = lax.axis_index("s") * NC + lax.axis_index("c")
        base = wid * CHUNK
        junk = jnp.full((LANES,), N, jnp.int32)
        u_v[pl.ds(FULL * LANES, LANES)] = junk
        v_v[pl.ds(FULL * LANES, LANES)] = junk
        pltpu.sync_copy(orc_hbm, orc_v.at[pl.ds(0, N)])
        pltpu.sync_copy(ei_hbm.at[0, pl.ds(base, CHUNK)],
                        u_v.at[pl.ds(0, CHUNK)])
        pltpu.sync_copy(ei_hbm.at[1, pl.ds(base, CHUNK)],
                        v_v.at[pl.ds(0, CHUNK)])

        zeros = jnp.zeros((LANES,), jnp.float32)

        def zero_body(i, carry):
            ms_v[pl.ds(i * LANES, LANES)] = zeros
            cn_v[pl.ds(i * LANES, LANES)] = zeros
            return carry

        lax.fori_loop(0, NP // LANES, zero_body, 0)

        ones = jnp.ones((LANES,), jnp.float32)

        def step(off):
            u16 = u_v[pl.ds(off, LANES)]
            v16 = v_v[pl.ds(off, LANES)]
            ou = plsc.load_gather(orc_v, [u16])
            ov = plsc.load_gather(orc_v, [v16])
            plsc.addupdate_scatter(ms_v, [u16], ov)
            plsc.addupdate_scatter(ms_v, [v16], ou)
            plsc.addupdate_scatter(cn_v, [u16], ones)
            plsc.addupdate_scatter(cn_v, [v16], ones)

        UNROLL = 4

        def body(i, carry):
            for k in range(UNROLL):
                step(i * (UNROLL * LANES) + k * LANES)
            return carry

        lax.fori_loop(0, FULL // UNROLL, body, 0)
        for k in range(FULL % UNROLL + 1):
            step((FULL // UNROLL * UNROLL + k) * LANES)

        pltpu.sync_copy(ms_v, msum_hbm.at[wid])
        pltpu.sync_copy(cn_v, cnt_hbm.at[wid])

    return sc_kernel(orc, ei_flat)


def _tc_body(orc_ref, ms_ref, cn_ref, w1_ref, b1_ref, w2_ref, b2_ref,
             g_ref, be_ref, out_ref):
    f32 = jnp.float32
    msum = jnp.sum(ms_ref[...], axis=0, keepdims=True)
    cnt = jnp.sum(cn_ref[...], axis=0, keepdims=True)
    nm = jnp.where(cnt > 0, msum / jnp.where(cnt > 0, cnt, 1.0), 0.0)
    orc = orc_ref[...]
    scale = 1.0 / (2.0 + 1e-8)
    no = jnp.clip((orc + 1.0) * scale, 0.0, 1.0)
    nn = jnp.clip((nm + 1.0) * scale, 0.0, 1.0)

    j = lax.broadcasted_iota(jnp.int32, (DC, 1), 0)
    base = jnp.where(j < DC // 2, no, nn)
    freq = (((j % (DC // 2)) // 2) + 1).astype(f32) * math.pi
    ang = base * freq
    phi = jnp.where(j % 2 == 0, jnp.sin(ang), jnp.cos(ang))

    h = lax.dot_general(w1_ref[...], phi, (((1,), (0,)), ((), ())),
                        preferred_element_type=f32) + b1_ref[...]
    h = jnp.maximum(h, 0.0)
    h2 = lax.dot_general(w2_ref[...], h, (((1,), (0,)), ((), ())),
                         preferred_element_type=f32) + b2_ref[...]
    mu = jnp.mean(h2, axis=0, keepdims=True)
    d = h2 - mu
    var = jnp.mean(d * d, axis=0, keepdims=True)
    ln = d / jnp.sqrt(var + 1e-5) * g_ref[...] + be_ref[...]
    outT = ln + phi
    eye = (lax.broadcasted_iota(jnp.int32, (DC, DC), 0)
           == lax.broadcasted_iota(jnp.int32, (DC, DC), 1)).astype(f32)
    out_ref[...] = lax.dot_general(outT, eye, (((0,), (0,)), ((), ())),
                                   preferred_element_type=f32)


def _tc_dense(orc_row, msum_p, cnt_p, W1, b1, W2, b2, gamma, beta,
              interpret=False):
    C = 2048
    return pl.pallas_call(
        _tc_body,
        grid=(NP // C,),
        in_specs=[
            pl.BlockSpec((1, C), lambda i: (0, i)),
            pl.BlockSpec((NW, C), lambda i: (0, i)),
            pl.BlockSpec((NW, C), lambda i: (0, i)),
            pl.BlockSpec((HID, DC), lambda i: (0, 0)),
            pl.BlockSpec((HID, 1), lambda i: (0, 0)),
            pl.BlockSpec((DC, HID), lambda i: (0, 0)),
            pl.BlockSpec((DC, 1), lambda i: (0, 0)),
            pl.BlockSpec((DC, 1), lambda i: (0, 0)),
            pl.BlockSpec((DC, 1), lambda i: (0, 0)),
        ],
        out_specs=pl.BlockSpec((C, DC), lambda i: (i, 0)),
        out_shape=jax.ShapeDtypeStruct((N, DC), jnp.float32),
        interpret=interpret,
    )(orc_row, msum_p, cnt_p, W1, b1, W2, b2, gamma, beta)


def kernel(node_orc, edge_index, W1, b1, W2, b2, gamma, beta):
    msum_p, cnt_p = _sc_scatter(node_orc, edge_index)
    return _tc_dense(node_orc.reshape(1, N), msum_p, cnt_p,
                     W1, b1.reshape(HID, 1), W2, b2.reshape(DC, 1),
                     gamma.reshape(DC, 1), beta.reshape(DC, 1))


# speedup vs baseline: 49.8781x; 1.0329x over previous
import functools
import math

import jax
import jax.numpy as jnp
from jax import lax
from jax.experimental import pallas as pl
from jax.experimental.pallas import tpu as pltpu
from jax.experimental.pallas import tpu_sc as plsc

N = 10000
E = 160000
DC = 16
HID = 32

NC = 2
NS = 16
NW = NC * NS
LANES = 16

NP = 10240
CHUNK = E // NW
FULL = CHUNK // LANES
TAIL = CHUNK - FULL * LANES
CCAP = (FULL + 1) * LANES


def _sc_scatter(orc, ei_flat):
    mesh = plsc.VectorSubcoreMesh(core_axis_name="c", subcore_axis_name="s")

    @functools.partial(
        pl.kernel,
        out_type=(
            jax.ShapeDtypeStruct((NW, NP), jnp.float32),
            jax.ShapeDtypeStruct((NW, NP), jnp.float32),
        ),
        mesh=mesh,
        compiler_params=pltpu.CompilerParams(
            use_tc_tiling_on_sc=False, needs_layout_passes=False),
        scratch_types=(
            pltpu.VMEM((NP,), jnp.float32),
            pltpu.VMEM((CCAP,), jnp.int32),
            pltpu.VMEM((CCAP,), jnp.int32),
            pltpu.VMEM((NP,), jnp.float32),
            pltpu.VMEM((NP,), jnp.float32),
        ),
    )
    def sc_kernel(orc_hbm, ei_hbm, msum_hbm, cnt_hbm,
                  orc_v, u_v, v_v, ms_v, cn_v):
        wid = lax.axis_index("s") * NC + lax.axis_index("c")
        base = wid * CHUNK
        junk = jnp.full((LANES,), N, jnp.int32)
        u_v[pl.ds(FULL * LANES, LANES)] = junk
        v_v[pl.ds(FULL * LANES, LANES)] = junk
        pltpu.sync_copy(orc_hbm, orc_v.at[pl.ds(0, N)])
        pltpu.sync_copy(ei_hbm.at[0, pl.ds(base, CHUNK)],
                        u_v.at[pl.ds(0, CHUNK)])
        pltpu.sync_copy(ei_hbm.at[1, pl.ds(base, CHUNK)],
                        v_v.at[pl.ds(0, CHUNK)])

        zeros = jnp.zeros((LANES,), jnp.float32)
        ZUNROLL = 8

        def zero_body(i, carry):
            for k in range(ZUNROLL):
                off = (i * ZUNROLL + k) * LANES
                ms_v[pl.ds(off, LANES)] = zeros
                cn_v[pl.ds(off, LANES)] = zeros
            return carry

        lax.fori_loop(0, NP // (LANES * ZUNROLL), zero_body, 0)

        ones = jnp.ones((LANES,), jnp.float32)

        def step(off):
            u16 = u_v[pl.ds(off, LANES)]
            v16 = v_v[pl.ds(off, LANES)]
            ou = plsc.load_gather(orc_v, [u16])
            ov = plsc.load_gather(orc_v, [v16])
            plsc.addupdate_scatter(ms_v, [u16], ov)
            plsc.addupdate_scatter(ms_v, [v16], ou)
            plsc.addupdate_scatter(cn_v, [u16], ones)
            plsc.addupdate_scatter(cn_v, [v16], ones)

        UNROLL = 8

        def body(i, carry):
            for k in range(UNROLL):
                step(i * (UNROLL * LANES) + k * LANES)
            return carry

        lax.fori_loop(0, FULL // UNROLL, body, 0)
        for k in range(FULL % UNROLL + 1):
            step((FULL // UNROLL * UNROLL + k) * LANES)

        pltpu.sync_copy(ms_v, msum_hbm.at[wid])
        pltpu.sync_copy(cn_v, cnt_hbm.at[wid])

    return sc_kernel(orc, ei_flat)


def _tc_body(orc_ref, ms_ref, cn_ref, w1_ref, b1_ref, w2_ref, b2_ref,
             g_ref, be_ref, out_ref):
    f32 = jnp.float32
    msum = jnp.sum(ms_ref[...], axis=0, keepdims=True)
    cnt = jnp.sum(cn_ref[...], axis=0, keepdims=True)
    nm = jnp.where(cnt > 0, msum / jnp.where(cnt > 0, cnt, 1.0), 0.0)
    orc = orc_ref[...]
    scale = 1.0 / (2.0 + 1e-8)
    no = jnp.clip((orc + 1.0) * scale, 0.0, 1.0)
    nn = jnp.clip((nm + 1.0) * scale, 0.0, 1.0)

    j = lax.broadcasted_iota(jnp.int32, (DC, 1), 0)
    base = jnp.where(j < DC // 2, no, nn)
    freq = (((j % (DC // 2)) // 2) + 1).astype(f32) * math.pi
    ang = base * freq
    phi = jnp.where(j % 2 == 0, jnp.sin(ang), jnp.cos(ang))

    h = lax.dot_general(w1_ref[...], phi, (((1,), (0,)), ((), ())),
                        preferred_element_type=f32) + b1_ref[...]
    h = jnp.maximum(h, 0.0)
    h2 = lax.dot_general(w2_ref[...], h, (((1,), (0,)), ((), ())),
                         preferred_element_type=f32) + b2_ref[...]
    mu = jnp.mean(h2, axis=0, keepdims=True)
    d = h2 - mu
    var = jnp.mean(d * d, axis=0, keepdims=True)
    ln = d / jnp.sqrt(var + 1e-5) * g_ref[...] + be_ref[...]
    outT = ln + phi
    eye = (lax.broadcasted_iota(jnp.int32, (DC, DC), 0)
           == lax.broadcasted_iota(jnp.int32, (DC, DC), 1)).astype(f32)
    out_ref[...] = lax.dot_general(outT, eye, (((0,), (0,)), ((), ())),
                                   preferred_element_type=f32)


def _tc_dense(orc_row, msum_p, cnt_p, W1, b1, W2, b2, gamma, beta,
              interpret=False):
    C = 2048
    return pl.pallas_call(
        _tc_body,
        grid=(NP // C,),
        in_specs=[
            pl.BlockSpec((1, C), lambda i: (0, i)),
            pl.BlockSpec((NW, C), lambda i: (0, i)),
            pl.BlockSpec((NW, C), lambda i: (0, i)),
            pl.BlockSpec((HID, DC), lambda i: (0, 0)),
            pl.BlockSpec((HID, 1), lambda i: (0, 0)),
            pl.BlockSpec((DC, HID), lambda i: (0, 0)),
            pl.BlockSpec((DC, 1), lambda i: (0, 0)),
            pl.BlockSpec((DC, 1), lambda i: (0, 0)),
            pl.BlockSpec((DC, 1), lambda i: (0, 0)),
        ],
        out_specs=pl.BlockSpec((C, DC), lambda i: (i, 0)),
        out_shape=jax.ShapeDtypeStruct((N, DC), jnp.float32),
        interpret=interpret,
    )(orc_row, msum_p, cnt_p, W1, b1, W2, b2, gamma, beta)


def kernel(node_orc, edge_index, W1, b1, W2, b2, gamma, beta):
    msum_p, cnt_p = _sc_scatter(node_orc, edge_index)
    return _tc_dense(node_orc.reshape(1, N), msum_p, cnt_p,
                     W1, b1.reshape(HID, 1), W2, b2.reshape(DC, 1),
                     gamma.reshape(DC, 1), beta.reshape(DC, 1))


# speedup vs baseline: 51.7991x; 1.0385x over previous
import functools
import math

import jax
import jax.numpy as jnp
from jax import lax
from jax.experimental import pallas as pl
from jax.experimental.pallas import tpu as pltpu
from jax.experimental.pallas import tpu_sc as plsc

N = 10000
E = 160000
DC = 16
HID = 32

NC = 2
NS = 16
NW = NC * NS
LANES = 16

NP = 10240
CHUNK = E // NW
FULL = CHUNK // LANES
TAIL = CHUNK - FULL * LANES
CCAP = (FULL + 1) * LANES


def _sc_scatter(orc, ei_flat):
    mesh = plsc.VectorSubcoreMesh(core_axis_name="c", subcore_axis_name="s")

    @functools.partial(
        pl.kernel,
        out_type=(
            jax.ShapeDtypeStruct((NW, NP), jnp.float32),
            jax.ShapeDtypeStruct((NW, NP), jnp.float32),
        ),
        mesh=mesh,
        compiler_params=pltpu.CompilerParams(
            use_tc_tiling_on_sc=False, needs_layout_passes=False),
        scratch_types=(
            pltpu.VMEM((NP,), jnp.float32),
            pltpu.VMEM((CCAP,), jnp.int32),
            pltpu.VMEM((CCAP,), jnp.int32),
            pltpu.VMEM((NP,), jnp.float32),
            pltpu.VMEM((NP,), jnp.float32),
            pltpu.SemaphoreType.DMA,
            pltpu.SemaphoreType.DMA,
            pltpu.SemaphoreType.DMA,
        ),
    )
    def sc_kernel(orc_hbm, ei_hbm, msum_hbm, cnt_hbm,
                  orc_v, u_v, v_v, ms_v, cn_v, sem0, sem1, sem2):
        wid = lax.axis_index("s") * NC + lax.axis_index("c")
        base = wid * CHUNK
        junk = jnp.full((LANES,), N, jnp.int32)
        u_v[pl.ds(FULL * LANES, LANES)] = junk
        v_v[pl.ds(FULL * LANES, LANES)] = junk
        cp0 = pltpu.async_copy(orc_hbm, orc_v.at[pl.ds(0, N)], sem0)
        cp1 = pltpu.async_copy(ei_hbm.at[0, pl.ds(base, CHUNK)],
                               u_v.at[pl.ds(0, CHUNK)], sem1)
        cp2 = pltpu.async_copy(ei_hbm.at[1, pl.ds(base, CHUNK)],
                               v_v.at[pl.ds(0, CHUNK)], sem2)

        zeros = jnp.zeros((LANES,), jnp.float32)
        ZUNROLL = 8

        def zero_body(i, carry):
            for k in range(ZUNROLL):
                off = (i * ZUNROLL + k) * LANES
                ms_v[pl.ds(off, LANES)] = zeros
                cn_v[pl.ds(off, LANES)] = zeros
            return carry

        lax.fori_loop(0, NP // (LANES * ZUNROLL), zero_body, 0)
        cp0.wait()
        cp1.wait()
        cp2.wait()

        ones = jnp.ones((LANES,), jnp.float32)

        def step(off):
            u16 = u_v[pl.ds(off, LANES)]
            v16 = v_v[pl.ds(off, LANES)]
            ou = plsc.load_gather(orc_v, [u16])
            ov = plsc.load_gather(orc_v, [v16])
            plsc.addupdate_scatter(ms_v, [u16], ov)
            plsc.addupdate_scatter(ms_v, [v16], ou)
            plsc.addupdate_scatter(cn_v, [u16], ones)
            plsc.addupdate_scatter(cn_v, [v16], ones)

        UNROLL = 8

        def body(i, carry):
            for k in range(UNROLL):
                step(i * (UNROLL * LANES) + k * LANES)
            return carry

        lax.fori_loop(0, FULL // UNROLL, body, 0)
        for k in range(FULL % UNROLL + 1):
            step((FULL // UNROLL * UNROLL + k) * LANES)

        st0 = pltpu.async_copy(ms_v, msum_hbm.at[wid], sem0)
        st1 = pltpu.async_copy(cn_v, cnt_hbm.at[wid], sem1)
        st0.wait()
        st1.wait()

    return sc_kernel(orc, ei_flat)


def _tc_body(orc_ref, ms_ref, cn_ref, w1_ref, b1_ref, w2_ref, b2_ref,
             g_ref, be_ref, out_ref):
    f32 = jnp.float32
    msum = jnp.sum(ms_ref[...], axis=0, keepdims=True)
    cnt = jnp.sum(cn_ref[...], axis=0, keepdims=True)
    nm = jnp.where(cnt > 0, msum / jnp.where(cnt > 0, cnt, 1.0), 0.0)
    orc = orc_ref[...]
    scale = 1.0 / (2.0 + 1e-8)
    no = jnp.clip((orc + 1.0) * scale, 0.0, 1.0)
    nn = jnp.clip((nm + 1.0) * scale, 0.0, 1.0)

    j = lax.broadcasted_iota(jnp.int32, (DC, 1), 0)
    base = jnp.where(j < DC // 2, no, nn)
    freq = (((j % (DC // 2)) // 2) + 1).astype(f32) * math.pi
    ang = base * freq
    phi = jnp.where(j % 2 == 0, jnp.sin(ang), jnp.cos(ang))

    h = lax.dot_general(w1_ref[...], phi, (((1,), (0,)), ((), ())),
                        preferred_element_type=f32) + b1_ref[...]
    h = jnp.maximum(h, 0.0)
    h2 = lax.dot_general(w2_ref[...], h, (((1,), (0,)), ((), ())),
                         preferred_element_type=f32) + b2_ref[...]
    mu = jnp.mean(h2, axis=0, keepdims=True)
    d = h2 - mu
    var = jnp.mean(d * d, axis=0, keepdims=True)
    ln = d / jnp.sqrt(var + 1e-5) * g_ref[...] + be_ref[...]
    outT = ln + phi
    eye = (lax.broadcasted_iota(jnp.int32, (DC, DC), 0)
           == lax.broadcasted_iota(jnp.int32, (DC, DC), 1)).astype(f32)
    out_ref[...] = lax.dot_general(outT, eye, (((0,), (0,)), ((), ())),
                                   preferred_element_type=f32)


def _tc_dense(orc_row, msum_p, cnt_p, W1, b1, W2, b2, gamma, beta,
              interpret=False):
    C = 2048
    return pl.pallas_call(
        _tc_body,
        grid=(NP // C,),
        in_specs=[
            pl.BlockSpec((1, C), lambda i: (0, i)),
            pl.BlockSpec((NW, C), lambda i: (0, i)),
            pl.BlockSpec((NW, C), lambda i: (0, i)),
            pl.BlockSpec((HID, DC), lambda i: (0, 0)),
            pl.BlockSpec((HID, 1), lambda i: (0, 0)),
            pl.BlockSpec((DC, HID), lambda i: (0, 0)),
            pl.BlockSpec((DC, 1), lambda i: (0, 0)),
            pl.BlockSpec((DC, 1), lambda i: (0, 0)),
            pl.BlockSpec((DC, 1), lambda i: (0, 0)),
        ],
        out_specs=pl.BlockSpec((C, DC), lambda i: (i, 0)),
        out_shape=jax.ShapeDtypeStruct((N, DC), jnp.float32),
        interpret=interpret,
    )(orc_row, msum_p, cnt_p, W1, b1, W2, b2, gamma, beta)


def kernel(node_orc, edge_index, W1, b1, W2, b2, gamma, beta):
    msum_p, cnt_p = _sc_scatter(node_orc, edge_index)
    return _tc_dense(node_orc.reshape(1, N), msum_p, cnt_p,
                     W1, b1.reshape(HID, 1), W2, b2.reshape(DC, 1),
                     gamma.reshape(DC, 1), beta.reshape(DC, 1))
